# baseline (speedup 1.0000x reference)
import functools

import jax
import jax.numpy as jnp
from jax import lax
from jax.experimental import pallas as pl
from jax.experimental.pallas import tpu as pltpu
from jax.experimental.pallas import tpu_sc as plsc

_V = 1000000
_D = 64
_B = 16384
_NC = 2
_NS = 16
_L = 16
_NW = _NC * _NS
_BPW = _B // _NW
_G = 128
_NCH = _BPW // _G


def _sc_glove(left_hbm, right_hbm, lemb_hbm, lbias_hbm, remb_hbm, rbias_hbm,
              out_hbm, idxl, idxr, gidl, gidr, lrows, rrows, lbv, rbv, tbuf,
              outv, sem):
    wid = lax.axis_index("s") * _NC + lax.axis_index("c")
    base = wid * _BPW

    pltpu.sync_copy(left_hbm.at[pl.ds(base, _BPW)], idxl)
    pltpu.sync_copy(right_hbm.at[pl.ds(base, _BPW)], idxr)

    for v in range(_BPW // _L):
        sl = pl.ds(v * _L, _L)
        gidl[sl] = lax.shift_right_logical(idxl[sl], 1)
        gidr[sl] = lax.shift_right_logical(idxr[sl], 1)

    iota = lax.iota(jnp.int32, _L)
    col0 = iota * _L

    for c in range(_NCH):
        csl = pl.ds(c * _G, _G)
        c1 = pltpu.async_copy(lemb_hbm.at[gidl.at[csl]], lrows, sem)
        c2 = pltpu.async_copy(remb_hbm.at[gidr.at[csl]], rrows, sem)
        c3 = pltpu.async_copy(lbias_hbm.at[idxl.at[csl]], lbv, sem)
        c4 = pltpu.async_copy(rbias_hbm.at[idxr.at[csl]], rbv, sem)
        c1.wait()
        c2.wait()
        c3.wait()
        c4.wait()

        def group(g, carry, c=c):
            hl = (idxl[pl.ds(c * _G + g * _L, _L)] & 1) * _D
            hr = (idxr[pl.ds(c * _G + g * _L, _L)] & 1) * _D
            for jj in range(_L):
                j = g * _L + jj
                ol = hl[jj]
                orr = hr[jj]
                p = (lrows[j, pl.ds(ol, _L)] * rrows[j, pl.ds(orr, _L)])
                for k in range(1, _D // _L):
                    p = p + (lrows[j, pl.ds(ol + k * _L, _L)] *
                             rrows[j, pl.ds(orr + k * _L, _L)])
                tbuf[pl.ds(jj * _L, _L)] = p
            acc = plsc.load_gather(tbuf, [col0])
            for t in range(1, _L):
                acc = acc + plsc.load_gather(tbuf, [col0 + t])
            gsl = pl.ds(g * _L, _L)
            outv[pl.ds(c * _G + g * _L, _L)] = acc + lbv[gsl] + rbv[gsl]
            return carry

        lax.fori_loop(0, _G // _L, group, 0)

    pltpu.sync_copy(outv, out_hbm.at[pl.ds(base, _BPW)])


@functools.cache
def _build():
    mesh = plsc.VectorSubcoreMesh(core_axis_name="c", subcore_axis_name="s")
    return pl.kernel(
        _sc_glove,
        mesh=mesh,
        compiler_params=pltpu.CompilerParams(
            needs_layout_passes=False, use_tc_tiling_on_sc=True),
        out_type=jax.ShapeDtypeStruct((_B,), jnp.float32),
        scratch_types=[
            pltpu.VMEM((_BPW,), jnp.int32),
            pltpu.VMEM((_BPW,), jnp.int32),
            pltpu.VMEM((_BPW,), jnp.int32),
            pltpu.VMEM((_BPW,), jnp.int32),
            pltpu.VMEM((_G, 2 * _D), jnp.float32),
            pltpu.VMEM((_G, 2 * _D), jnp.float32),
            pltpu.VMEM((_G,), jnp.float32),
            pltpu.VMEM((_G,), jnp.float32),
            pltpu.VMEM((_L * _L,), jnp.float32),
            pltpu.VMEM((_BPW,), jnp.float32),
            pltpu.SemaphoreType.DMA,
        ],
    )


def kernel(left, right, l_emb, l_bias, r_emb, r_bias):
    ar = jnp.arange(_V, dtype=jnp.int32)
    return _build()(left.astype(jnp.int32), right.astype(jnp.int32),
                    l_emb.reshape(_V // 2, 2 * _D), l_bias[ar, 0],
                    r_emb.reshape(_V // 2, 2 * _D), r_bias[ar, 0])


# speedup vs baseline: 1.1412x; 1.1412x over previous
import functools

import jax
import jax.numpy as jnp
from jax import lax
from jax.experimental import pallas as pl
from jax.experimental.pallas import tpu as pltpu
from jax.experimental.pallas import tpu_sc as plsc

_V = 1000000
_D = 64
_B = 16384
_NC = 2
_NS = 16
_L = 16
_NW = _NC * _NS
_BPW = _B // _NW
_G = 128
_NCH = _BPW // _G


def _sc_glove(left_hbm, right_hbm, lemb_hbm, lbias_hbm, remb_hbm, rbias_hbm,
              out_hbm, idxl, idxr, gidl, gidr, lrows, rrows, lbv, rbv, tbuf,
              outv, sem):
    wid = lax.axis_index("s") * _NC + lax.axis_index("c")
    base = wid * _BPW

    pltpu.sync_copy(left_hbm.at[pl.ds(base, _BPW)], idxl)
    pltpu.sync_copy(right_hbm.at[pl.ds(base, _BPW)], idxr)

    for v in range(_BPW // _L):
        sl = pl.ds(v * _L, _L)
        gidl[sl] = lax.shift_right_logical(idxl[sl], 1)
        gidr[sl] = lax.shift_right_logical(idxr[sl], 1)

    iota = lax.iota(jnp.int32, _L)
    col0 = iota * _L

    for c in range(_NCH):
        csl = pl.ds(c * _G, _G)
        c1 = pltpu.async_copy(lemb_hbm.at[gidl.at[csl]], lrows, sem)
        c2 = pltpu.async_copy(remb_hbm.at[gidr.at[csl]], rrows, sem)
        c3 = pltpu.async_copy(lbias_hbm.at[idxl.at[csl]], lbv, sem)
        c4 = pltpu.async_copy(rbias_hbm.at[idxr.at[csl]], rbv, sem)
        c1.wait()
        c2.wait()
        c3.wait()
        c4.wait()

        def group(g, carry, c=c):
            hl = (idxl[pl.ds(c * _G + g * _L, _L)] & 1) * _D
            hr = (idxr[pl.ds(c * _G + g * _L, _L)] & 1) * _D
            for jj in range(_L):
                j = g * _L + jj
                ol = hl[jj]
                orr = hr[jj]
                p = (lrows[j, pl.ds(ol, _L)] * rrows[j, pl.ds(orr, _L)])
                for k in range(1, _D // _L):
                    p = p + (lrows[j, pl.ds(ol + k * _L, _L)] *
                             rrows[j, pl.ds(orr + k * _L, _L)])
                tbuf[pl.ds(jj * _L, _L)] = p
            acc = plsc.load_gather(tbuf, [col0])
            for t in range(1, _L):
                acc = acc + plsc.load_gather(tbuf, [col0 + t])
            gsl = pl.ds(g * _L, _L)
            outv[pl.ds(c * _G + g * _L, _L)] = acc + lbv[gsl] + rbv[gsl]
            return carry

        lax.fori_loop(0, _G // _L, group, 0)

    pltpu.sync_copy(outv, out_hbm.at[pl.ds(base, _BPW)])


@functools.cache
def _build():
    mesh = plsc.VectorSubcoreMesh(core_axis_name="c", subcore_axis_name="s")
    return pl.kernel(
        _sc_glove,
        mesh=mesh,
        compiler_params=pltpu.CompilerParams(
            needs_layout_passes=False, use_tc_tiling_on_sc=True),
        out_type=jax.ShapeDtypeStruct((_B,), jnp.float32),
        scratch_types=[
            pltpu.VMEM((_BPW,), jnp.int32),
            pltpu.VMEM((_BPW,), jnp.int32),
            pltpu.VMEM((_BPW,), jnp.int32),
            pltpu.VMEM((_BPW,), jnp.int32),
            pltpu.VMEM((_G, 2 * _D), jnp.float32),
            pltpu.VMEM((_G, 2 * _D), jnp.float32),
            pltpu.VMEM((_G,), jnp.float32),
            pltpu.VMEM((_G,), jnp.float32),
            pltpu.VMEM((_L * _L,), jnp.float32),
            pltpu.VMEM((_BPW,), jnp.float32),
            pltpu.SemaphoreType.DMA,
        ],
    )


def kernel(left, right, l_emb, l_bias, r_emb, r_bias):
    zero = lax.optimization_barrier(jnp.float32(0.0))
    return _build()(left.astype(jnp.int32), right.astype(jnp.int32),
                    l_emb.reshape(_V // 2, 2 * _D),
                    (l_bias + zero).reshape(-1),
                    r_emb.reshape(_V // 2, 2 * _D),
                    (r_bias + zero).reshape(-1))
